# Initial kernel scaffold; baseline (speedup 1.0000x reference)
import jax
import jax.numpy as jnp
from jax.experimental import pallas as pl


def kernel(x, adj, W, W_ih, W_hh, b_ih, b_hh):
    raise NotImplementedError("write your pallas kernel here")


# speedup vs baseline: 1.2451x; 1.2451x over previous
import functools

import jax
import jax.numpy as jnp
from jax import lax
from jax.experimental import pallas as pl

NUM_LAYERS = 3


def _ggnn_kernel(x_ref, adj_ref, w_ref, wih_ref, whh_ref, bih_ref, bhh_ref,
                 out_ref, *, num_layers, d):
    h = x_ref[0]
    A = adj_ref[0].astype(jnp.float32)
    b_ih = bih_ref[:, :]
    b_hh = bhh_ref[:, :]
    for l in range(num_layers):
        m = lax.dot_general(h, w_ref[l], (((1,), (0,)), ((), ())),
                            preferred_element_type=jnp.float32)
        agg = lax.dot_general(A, m, (((0,), (0,)), ((), ())),
                              preferred_element_type=jnp.float32)
        gi = lax.dot_general(agg, wih_ref[:, :], (((1,), (0,)), ((), ())),
                             preferred_element_type=jnp.float32) + b_ih
        gh = lax.dot_general(h, whh_ref[:, :], (((1,), (0,)), ((), ())),
                             preferred_element_type=jnp.float32) + b_hh
        r = jax.nn.sigmoid(gi[:, 0:d] + gh[:, 0:d])
        z = jax.nn.sigmoid(gi[:, d:2 * d] + gh[:, d:2 * d])
        n = jnp.tanh(gi[:, 2 * d:3 * d] + r * gh[:, 2 * d:3 * d])
        h = (1.0 - z) * n + z * h
    out_ref[0] = h


def kernel(x, adj, W, W_ih, W_hh, b_ih, b_hh):
    B, N, D = x.shape
    num_layers = W.shape[0]
    W_ih_t = W_ih.T
    W_hh_t = W_hh.T
    b_ih2 = b_ih.reshape(1, 3 * D)
    b_hh2 = b_hh.reshape(1, 3 * D)
    grid = (B,)
    return pl.pallas_call(
        functools.partial(_ggnn_kernel, num_layers=num_layers, d=D),
        grid=grid,
        in_specs=[
            pl.BlockSpec((1, N, D), lambda b: (b, 0, 0)),
            pl.BlockSpec((1, N, N), lambda b: (b, 0, 0)),
            pl.BlockSpec((num_layers, D, D), lambda b: (0, 0, 0)),
            pl.BlockSpec((D, 3 * D), lambda b: (0, 0)),
            pl.BlockSpec((D, 3 * D), lambda b: (0, 0)),
            pl.BlockSpec((1, 3 * D), lambda b: (0, 0)),
            pl.BlockSpec((1, 3 * D), lambda b: (0, 0)),
        ],
        out_specs=pl.BlockSpec((1, N, D), lambda b: (b, 0, 0)),
        out_shape=jax.ShapeDtypeStruct((B, N, D), jnp.float32),
    )(x, adj, W, W_ih_t, W_hh_t, b_ih2, b_hh2)
